# baseline (speedup 1.0000x reference)
import functools

import jax
import jax.numpy as jnp
from jax import lax
from jax.experimental import pallas as pl
from jax.experimental.pallas import tpu as pltpu
from jax.experimental.pallas import tpu_sc as plsc

_N = 10000
_D = 128
_E = 160000
_NPAD = 10240
_NC = 2
_NS = 16
_L = 16
_CH = 80
_TE = _CH * 128
_EP = _TE * _NS
_SL = _EP // 8
_GC = 16

_mesh = plsc.VectorSubcoreMesh(
    core_axis_name="c", subcore_axis_name="s", num_cores=_NC, num_subcores=_NS
)


def _rsqrt16(d):
    i = plsc.bitcast(d, jnp.int32)
    magic = jnp.full((_L,), 0x5F3759DF, jnp.int32)
    one = jnp.full((_L,), 1, jnp.int32)
    y = plsc.bitcast(magic - lax.shift_right_logical(i, one), jnp.float32)
    h = 0.5 * d
    y = y * (1.5 - h * y * y)
    y = y * (1.5 - h * y * y)
    y = y * (1.5 - h * y * y)
    return jnp.where(d > 0.0, y, 0.0)


def _deg_body(idx_hbm, norms_hbm, deg, idxb, accb, tmpb, part):
    c = lax.axis_index("c")
    s = lax.axis_index("s")
    a = 2 * c + s // 8
    slot = s % 8
    zeros16 = jnp.zeros((_L,), jnp.float32)
    ones16 = jnp.ones((_L,), jnp.float32)

    def z(k, _):
        deg[pl.ds(k * _L, _L)] = zeros16
        return 0

    lax.fori_loop(0, _NPAD // _L, z, 0)

    pltpu.sync_copy(idx_hbm.at[pl.ds(a * _EP + slot * _SL, _SL)], idxb)

    def cnt(k, _):
        iv = idxb[pl.ds(k * _L, _L)]
        plsc.addupdate_scatter(deg, [iv], ones16)
        return 0

    lax.fori_loop(0, _SL // _L, cnt, 0)

    pltpu.sync_copy(deg, part.at[s])
    plsc.subcore_barrier()

    nseg = _NPAD // 8
    base = slot * nseg
    row0 = (s // 8) * 8
    pltpu.sync_copy(part.at[row0, pl.ds(base, nseg)], accb)

    def red(r, _):
        pltpu.sync_copy(part.at[row0 + r, pl.ds(base, nseg)], tmpb)

        def add(k, __):
            accb[pl.ds(k * _L, _L)] = accb[pl.ds(k * _L, _L)] + tmpb[pl.ds(k * _L, _L)]
            return 0

        lax.fori_loop(0, nseg // _L, add, 0)
        return 0

    lax.fori_loop(1, 8, red, 0)

    def nrm(k, _):
        accb[pl.ds(k * _L, _L)] = _rsqrt16(accb[pl.ds(k * _L, _L)])
        return 0

    lax.fori_loop(0, nseg // _L, nrm, 0)
    pltpu.sync_copy(accb, norms_hbm.at[pl.ds(a * _NPAD + base, nseg)])


def _agg_body(xs_hbm, src_hbm, dst_hbm, out_hbm, srcb, dstb, rows0, rows1, acc, gs0, gs1, ss0, ss1):
    c = lax.axis_index("c")
    s = lax.axis_index("s")
    base_rows = c * (_NS * _CH) + s * _CH

    zeros16 = jnp.zeros((_L,), jnp.float32)

    def z(t, _):
        rows0[t // 8, pl.ds((t % 8) * _L, _L)] = zeros16
        return 0

    lax.fori_loop(0, 1024, z, 0)
    nrows = _NPAD // _NS

    def zc(k, _):
        pltpu.sync_copy(rows0, acc.at[pl.ds(s * nrows + k * 128, 128)])
        return 0

    lax.fori_loop(0, nrows // 128, zc, 0)
    plsc.subcore_barrier()

    def g_start(j, buf, sem):
        pltpu.async_copy(xs_hbm.at[srcb.at[j]], buf, sem)

    def g_wait(j, buf, sem):
        pltpu.make_async_copy(xs_hbm.at[srcb.at[j]], buf, sem).wait()

    def s_start(j, buf, sem):
        pltpu.async_copy(buf, acc.at[dstb.at[j]], sem, add=True)

    def s_wait(j, buf, sem):
        pltpu.make_async_copy(buf, acc.at[dstb.at[j]], sem).wait()

    def group(g, _):
        pltpu.sync_copy(src_hbm.at[pl.ds(base_rows + g * _GC, _GC)], srcb)
        pltpu.sync_copy(dst_hbm.at[pl.ds(base_rows + g * _GC, _GC)], dstb)
        g_start(0, rows0, gs0)

        def body(i, __):
            j0 = 2 * i
            g_wait(j0, rows0, gs0)
            s_start(j0, rows0, ss0)

            @pl.when(i > 0)
            def _():
                s_wait(j0 - 1, rows1, ss1)

            g_start(j0 + 1, rows1, gs1)
            g_wait(j0 + 1, rows1, gs1)
            s_start(j0 + 1, rows1, ss1)
            s_wait(j0, rows0, ss0)

            @pl.when(i < _GC // 2 - 1)
            def _():
                g_start(j0 + 2, rows0, gs0)

            return 0

        lax.fori_loop(0, _GC // 2, body, 0)
        s_wait(_GC - 1, rows1, ss1)
        return 0

    lax.fori_loop(0, _CH // _GC, group, 0)
    plsc.subcore_barrier()

    def dr(k, _):
        pltpu.sync_copy(acc.at[pl.ds(s * nrows + k * 128, 128)], rows0)
        pltpu.sync_copy(rows0, out_hbm.at[pl.ds(c * _NPAD + s * nrows + k * 128, 128)])
        return 0

    lax.fori_loop(0, nrows // 128, dr, 0)


_sc_params = pltpu.CompilerParams(needs_layout_passes=False)

_deg_kernel = pl.kernel(
    _deg_body,
    out_type=jax.ShapeDtypeStruct((4 * _NPAD,), jnp.float32),
    mesh=_mesh,
    compiler_params=_sc_params,
    scratch_types=[
        pltpu.VMEM((_NPAD,), jnp.float32),
        pltpu.VMEM((_SL,), jnp.int32),
        pltpu.VMEM((_NPAD // 8,), jnp.float32),
        pltpu.VMEM((_NPAD // 8,), jnp.float32),
        pltpu.VMEM_SHARED((_NS, _NPAD), jnp.float32),
    ],
)

_agg_kernel = pl.kernel(
    _agg_body,
    out_type=jax.ShapeDtypeStruct((2 * _NPAD, _D), jnp.float32),
    mesh=_mesh,
    compiler_params=_sc_params,
    scratch_types=[
        pltpu.VMEM((_GC, 128), jnp.int32),
        pltpu.VMEM((_GC, 128), jnp.int32),
        pltpu.VMEM((128, _D), jnp.float32),
        pltpu.VMEM((128, _D), jnp.float32),
        pltpu.VMEM_SHARED((_NPAD, _D), jnp.float32),
        pltpu.SemaphoreType.DMA,
        pltpu.SemaphoreType.DMA,
        pltpu.SemaphoreType.DMA,
        pltpu.SemaphoreType.DMA,
    ],
)


def _col(row):
    ones = jnp.ones((1, 128), jnp.float32)
    return lax.dot_general(row, ones, (((0,), (0,)), ((), ())),
                           preferred_element_type=jnp.float32)


def _scale_body(x_ref, n_ref, o_ref):
    o_ref[...] = x_ref[...] * _col(n_ref[0])


def _final_body(af_ref, al_ref, nf_ref, nl_ref, wf_ref, wl_ref, bf_ref, bl_ref, o_ref):
    accf = af_ref[...] * _col(nf_ref[0])
    accl = al_ref[...] * _col(nl_ref[0])
    h = (jnp.dot(accf, wf_ref[...], preferred_element_type=jnp.float32)
         + jnp.dot(accl, wl_ref[...], preferred_element_type=jnp.float32)
         + bf_ref[...] + bl_ref[...])
    o_ref[...] = jnp.maximum(h, 0.0)


_MB = 80


def _scale_call(x_pad, norms2d):
    return pl.pallas_call(
        _scale_body,
        grid=(2 * _MB,),
        in_specs=[
            pl.BlockSpec((128, _D), lambda i: (i % _MB, 0)),
            pl.BlockSpec((1, 1, 128), lambda i: (2 * _MB * (i // _MB) + i % _MB, 0, 0)),
        ],
        out_specs=pl.BlockSpec((128, _D), lambda i: (i, 0)),
        out_shape=jax.ShapeDtypeStruct((2 * _NPAD, _D), jnp.float32),
    )(x_pad, norms2d)


_OB = 79


def _final_call(agg, norms2d, wf, wl, bf2, bl2):
    return pl.pallas_call(
        _final_body,
        grid=(_OB,),
        in_specs=[
            pl.BlockSpec((128, _D), lambda i: (i, 0)),
            pl.BlockSpec((128, _D), lambda i: (i + _MB, 0)),
            pl.BlockSpec((1, 1, 128), lambda i: (_MB + i, 0, 0)),
            pl.BlockSpec((1, 1, 128), lambda i: (3 * _MB + i, 0, 0)),
            pl.BlockSpec((_D, _D), lambda i: (0, 0)),
            pl.BlockSpec((_D, _D), lambda i: (0, 0)),
            pl.BlockSpec((1, 128), lambda i: (0, 0)),
            pl.BlockSpec((1, 128), lambda i: (0, 0)),
        ],
        out_specs=pl.BlockSpec((128, _D), lambda i: (i, 0)),
        out_shape=jax.ShapeDtypeStruct((_OB * 128, _D), jnp.float32),
    )(agg, agg, norms2d, norms2d, wf, wl, bf2, bl2)


def kernel(x, edge_index_follows, edge_index_likes, W_follows, b_follows, W_likes, b_likes):
    i32 = jnp.int32
    x_pad = jnp.concatenate([x, jnp.zeros((_NPAD - _N, _D), x.dtype)], axis=0)
    pad = jnp.full((_EP - _E,), _N, i32)
    srcf = jnp.concatenate([edge_index_follows[0], pad])
    dstf = jnp.concatenate([edge_index_follows[1], pad])
    srcl = jnp.concatenate([edge_index_likes[0], pad])
    dstl = jnp.concatenate([edge_index_likes[1], pad])

    idx_all = jnp.concatenate([srcf, dstf, srcl, dstl])
    norms = _deg_kernel(idx_all)
    norms2d = norms.reshape(4 * _MB, 1, 128)

    xs = _scale_call(x_pad, norms2d)

    src_c = jnp.concatenate([srcf, srcl + _NPAD]).reshape(2 * _NS * _CH, 128)
    dst_c = jnp.concatenate([dstf, dstl]).reshape(2 * _NS * _CH, 128)
    agg = _agg_kernel(xs, src_c, dst_c)

    out = _final_call(agg, norms2d, W_follows, W_likes,
                      b_follows.reshape(1, _D), b_likes.reshape(1, _D))
    return out[:_N]


# speedup vs baseline: 1.3124x; 1.3124x over previous
import functools

import jax
import jax.numpy as jnp
from jax import lax
from jax.experimental import pallas as pl
from jax.experimental.pallas import tpu as pltpu
from jax.experimental.pallas import tpu_sc as plsc

_N = 10000
_D = 128
_E = 160000
_NPAD = 10240
_NC = 2
_NS = 16
_L = 16
_CH = 80
_TE = _CH * 128
_EP = _TE * _NS
_SL = _EP // 8
_GC = 16

_mesh = plsc.VectorSubcoreMesh(
    core_axis_name="c", subcore_axis_name="s", num_cores=_NC, num_subcores=_NS
)


def _rsqrt16(d):
    i = plsc.bitcast(d, jnp.int32)
    magic = jnp.full((_L,), 0x5F3759DF, jnp.int32)
    one = jnp.full((_L,), 1, jnp.int32)
    y = plsc.bitcast(magic - lax.shift_right_logical(i, one), jnp.float32)
    h = 0.5 * d
    y = y * (1.5 - h * y * y)
    y = y * (1.5 - h * y * y)
    y = y * (1.5 - h * y * y)
    return jnp.where(d > 0.0, y, 0.0)


def _deg_body(idx_hbm, norms_hbm, deg, idxb, accb, tmpb, part):
    c = lax.axis_index("c")
    s = lax.axis_index("s")
    a = 2 * c + s // 8
    slot = s % 8
    zeros16 = jnp.zeros((_L,), jnp.float32)
    ones16 = jnp.ones((_L,), jnp.float32)

    def z(k, _):
        deg[pl.ds(k * _L, _L)] = zeros16
        return 0

    lax.fori_loop(0, _NPAD // _L, z, 0)

    pltpu.sync_copy(idx_hbm.at[pl.ds(a * _EP + slot * _SL, _SL)], idxb)

    def cnt(k, _):
        iv = idxb[pl.ds(k * _L, _L)]
        plsc.addupdate_scatter(deg, [iv], ones16)
        return 0

    lax.fori_loop(0, _SL // _L, cnt, 0)

    pltpu.sync_copy(deg, part.at[s])
    plsc.subcore_barrier()

    nseg = _NPAD // 8
    base = slot * nseg
    row0 = (s // 8) * 8
    pltpu.sync_copy(part.at[row0, pl.ds(base, nseg)], accb)

    def red(r, _):
        pltpu.sync_copy(part.at[row0 + r, pl.ds(base, nseg)], tmpb)

        def add(k, __):
            accb[pl.ds(k * _L, _L)] = accb[pl.ds(k * _L, _L)] + tmpb[pl.ds(k * _L, _L)]
            return 0

        lax.fori_loop(0, nseg // _L, add, 0)
        return 0

    lax.fori_loop(1, 8, red, 0)

    def nrm(k, _):
        accb[pl.ds(k * _L, _L)] = _rsqrt16(accb[pl.ds(k * _L, _L)])
        return 0

    lax.fori_loop(0, nseg // _L, nrm, 0)
    pltpu.sync_copy(accb, norms_hbm.at[pl.ds(a * _NPAD + base, nseg)])


def _agg_body(x_hbm, norms_hbm, src_hbm, dst_hbm, xs_hbm, out_hbm,
              srcb, dstb, rows0, rows1, nsb, acc, gs0, gs1, ss0, ss1):
    c = lax.axis_index("c")
    s = lax.axis_index("s")
    base_rows = c * (_NS * _CH) + s * _CH
    nrows = _NPAD // _NS

    pltpu.sync_copy(norms_hbm.at[pl.ds(2 * c * _NPAD + s * nrows, nrows)], nsb)

    def sc_chunk(k, _):
        pltpu.sync_copy(x_hbm.at[pl.ds(s * nrows + k * 128, 128)], rows0)

        def rowgrp(g, __):
            wv = nsb[pl.ds(k * 128 + g * _L, _L)]
            for j in range(_L):
                r = g * _L + j
                w = wv[j]
                for v in range(_D // _L):
                    rows0[r, pl.ds(v * _L, _L)] = rows0[r, pl.ds(v * _L, _L)] * w
            return 0

        lax.fori_loop(0, 128 // _L, rowgrp, 0)
        pltpu.sync_copy(rows0, xs_hbm.at[pl.ds(c * _NPAD + s * nrows + k * 128, 128)])
        return 0

    lax.fori_loop(0, nrows // 128, sc_chunk, 0)

    zeros16 = jnp.zeros((_L,), jnp.float32)

    def z(t, _):
        rows0[t // 8, pl.ds((t % 8) * _L, _L)] = zeros16
        return 0

    lax.fori_loop(0, 1024, z, 0)

    def zc(k, _):
        pltpu.sync_copy(rows0, acc.at[pl.ds(s * nrows + k * 128, 128)])
        return 0

    lax.fori_loop(0, nrows // 128, zc, 0)
    plsc.subcore_barrier()

    def g_start(j, buf, sem):
        pltpu.async_copy(xs_hbm.at[srcb.at[j]], buf, sem)

    def g_wait(j, buf, sem):
        pltpu.make_async_copy(xs_hbm.at[srcb.at[j]], buf, sem).wait()

    def s_start(j, buf, sem):
        pltpu.async_copy(buf, acc.at[dstb.at[j]], sem, add=True)

    def s_wait(j, buf, sem):
        pltpu.make_async_copy(buf, acc.at[dstb.at[j]], sem).wait()

    def group(g, _):
        pltpu.sync_copy(src_hbm.at[pl.ds(base_rows + g * _GC, _GC)], srcb)
        pltpu.sync_copy(dst_hbm.at[pl.ds(base_rows + g * _GC, _GC)], dstb)
        g_start(0, rows0, gs0)

        def body(i, __):
            j0 = 2 * i
            g_wait(j0, rows0, gs0)
            s_start(j0, rows0, ss0)

            @pl.when(i > 0)
            def _():
                s_wait(j0 - 1, rows1, ss1)

            g_start(j0 + 1, rows1, gs1)
            g_wait(j0 + 1, rows1, gs1)
            s_start(j0 + 1, rows1, ss1)
            s_wait(j0, rows0, ss0)

            @pl.when(i < _GC // 2 - 1)
            def _():
                g_start(j0 + 2, rows0, gs0)

            return 0

        lax.fori_loop(0, _GC // 2, body, 0)
        s_wait(_GC - 1, rows1, ss1)
        return 0

    lax.fori_loop(0, _CH // _GC, group, 0)
    plsc.subcore_barrier()

    def dr(k, _):
        pltpu.sync_copy(acc.at[pl.ds(s * nrows + k * 128, 128)], rows0)
        pltpu.sync_copy(rows0, out_hbm.at[pl.ds(c * _NPAD + s * nrows + k * 128, 128)])
        return 0

    lax.fori_loop(0, nrows // 128, dr, 0)


_sc_params = pltpu.CompilerParams(needs_layout_passes=False)

_deg_kernel = pl.kernel(
    _deg_body,
    out_type=jax.ShapeDtypeStruct((4 * _NPAD,), jnp.float32),
    mesh=_mesh,
    compiler_params=_sc_params,
    scratch_types=[
        pltpu.VMEM((_NPAD,), jnp.float32),
        pltpu.VMEM((_SL,), jnp.int32),
        pltpu.VMEM((_NPAD // 8,), jnp.float32),
        pltpu.VMEM((_NPAD // 8,), jnp.float32),
        pltpu.VMEM_SHARED((_NS, _NPAD), jnp.float32),
    ],
)

_agg_kernel = pl.kernel(
    _agg_body,
    out_type=(
        jax.ShapeDtypeStruct((2 * _NPAD, _D), jnp.float32),
        jax.ShapeDtypeStruct((2 * _NPAD, _D), jnp.float32),
    ),
    mesh=_mesh,
    compiler_params=_sc_params,
    scratch_types=[
        pltpu.VMEM((_GC, 128), jnp.int32),
        pltpu.VMEM((_GC, 128), jnp.int32),
        pltpu.VMEM((128, _D), jnp.float32),
        pltpu.VMEM((128, _D), jnp.float32),
        pltpu.VMEM((_NPAD // _NS,), jnp.float32),
        pltpu.VMEM_SHARED((_NPAD, _D), jnp.float32),
        pltpu.SemaphoreType.DMA,
        pltpu.SemaphoreType.DMA,
        pltpu.SemaphoreType.DMA,
        pltpu.SemaphoreType.DMA,
    ],
)


_BR = 1024
_SUB = _BR // 128


def _colmat(row):
    ones = jnp.ones((1, 128), jnp.float32)
    return lax.dot_general(row, ones, (((0,), (0,)), ((), ())),
                           preferred_element_type=jnp.float32)


def _final_body(af_ref, al_ref, nf_ref, nl_ref, wf_ref, wl_ref, bf_ref, bl_ref, o_ref):
    mf = jnp.concatenate([_colmat(nf_ref[0, r:r + 1, :]) for r in range(_SUB)], axis=0)
    ml = jnp.concatenate([_colmat(nl_ref[0, r:r + 1, :]) for r in range(_SUB)], axis=0)
    h = (jnp.dot(af_ref[...] * mf, wf_ref[...], preferred_element_type=jnp.float32)
         + jnp.dot(al_ref[...] * ml, wl_ref[...], preferred_element_type=jnp.float32)
         + bf_ref[...] + bl_ref[...])
    o_ref[...] = jnp.maximum(h, 0.0)


_GB = _NPAD // _BR


def _final_call(agg, norms3, wf, wl, bf2, bl2):
    return pl.pallas_call(
        _final_body,
        grid=(_GB,),
        in_specs=[
            pl.BlockSpec((_BR, _D), lambda i: (i, 0)),
            pl.BlockSpec((_BR, _D), lambda i: (i + _GB, 0)),
            pl.BlockSpec((1, _SUB, 128), lambda i: (_GB + i, 0, 0)),
            pl.BlockSpec((1, _SUB, 128), lambda i: (3 * _GB + i, 0, 0)),
            pl.BlockSpec((_D, _D), lambda i: (0, 0)),
            pl.BlockSpec((_D, _D), lambda i: (0, 0)),
            pl.BlockSpec((1, 128), lambda i: (0, 0)),
            pl.BlockSpec((1, 128), lambda i: (0, 0)),
        ],
        out_specs=pl.BlockSpec((_BR, _D), lambda i: (i, 0)),
        out_shape=jax.ShapeDtypeStruct((_N, _D), jnp.float32),
    )(agg, agg, norms3, norms3, wf, wl, bf2, bl2)


def kernel(x, edge_index_follows, edge_index_likes, W_follows, b_follows, W_likes, b_likes):
    i32 = jnp.int32
    x_pad = jnp.concatenate([x, jnp.zeros((_NPAD - _N, _D), x.dtype)], axis=0)
    pad = jnp.full((_EP - _E,), _N, i32)
    srcf = jnp.concatenate([edge_index_follows[0], pad])
    dstf = jnp.concatenate([edge_index_follows[1], pad])
    srcl = jnp.concatenate([edge_index_likes[0], pad])
    dstl = jnp.concatenate([edge_index_likes[1], pad])

    idx_all = jnp.concatenate([srcf, dstf, srcl, dstl])
    norms = _deg_kernel(idx_all)

    src_c = jnp.concatenate([srcf, srcl + _NPAD]).reshape(2 * _NS * _CH, 128)
    dst_c = jnp.concatenate([dstf, dstl]).reshape(2 * _NS * _CH, 128)
    _, agg = _agg_kernel(x_pad, norms, src_c, dst_c)

    out = _final_call(agg, norms.reshape(4 * _GB, _SUB, 128), W_follows, W_likes,
                      b_follows.reshape(1, _D), b_likes.reshape(1, _D))
    return out


# speedup vs baseline: 1.3968x; 1.0643x over previous
import functools

import jax
import jax.numpy as jnp
from jax import lax
from jax.experimental import pallas as pl
from jax.experimental.pallas import tpu as pltpu
from jax.experimental.pallas import tpu_sc as plsc

_N = 10000
_D = 128
_E = 160000
_NPAD = 10240
_NC = 2
_NS = 16
_L = 16
_CH = 80
_TE = _CH * 128
_EP = _TE * _NS
_SL = _EP // 8
_GC = 40

_mesh = plsc.VectorSubcoreMesh(
    core_axis_name="c", subcore_axis_name="s", num_cores=_NC, num_subcores=_NS
)


def _rsqrt16(d):
    i = plsc.bitcast(d, jnp.int32)
    magic = jnp.full((_L,), 0x5F3759DF, jnp.int32)
    one = jnp.full((_L,), 1, jnp.int32)
    y = plsc.bitcast(magic - lax.shift_right_logical(i, one), jnp.float32)
    h = 0.5 * d
    y = y * (1.5 - h * y * y)
    y = y * (1.5 - h * y * y)
    y = y * (1.5 - h * y * y)
    return jnp.where(d > 0.0, y, 0.0)


def _deg_body(idx_hbm, norms_hbm, deg, idxb, accb, tmpb, part):
    c = lax.axis_index("c")
    s = lax.axis_index("s")
    a = 2 * c + s // 8
    slot = s % 8
    zeros16 = jnp.zeros((_L,), jnp.float32)
    ones16 = jnp.ones((_L,), jnp.float32)

    def z(k, _):
        deg[pl.ds(k * _L, _L)] = zeros16
        return 0

    lax.fori_loop(0, _NPAD // _L, z, 0)

    pltpu.sync_copy(idx_hbm.at[pl.ds(a * _EP + slot * _SL, _SL)], idxb)

    def cnt(k, _):
        iv = idxb[pl.ds(k * _L, _L)]
        plsc.addupdate_scatter(deg, [iv], ones16)
        return 0

    lax.fori_loop(0, _SL // _L, cnt, 0)

    pltpu.sync_copy(deg, part.at[s])
    plsc.subcore_barrier()

    nseg = _NPAD // 8
    base = slot * nseg
    row0 = (s // 8) * 8
    pltpu.sync_copy(part.at[row0, pl.ds(base, nseg)], accb)

    def red(r, _):
        pltpu.sync_copy(part.at[row0 + r, pl.ds(base, nseg)], tmpb)

        def add(k, __):
            accb[pl.ds(k * _L, _L)] = accb[pl.ds(k * _L, _L)] + tmpb[pl.ds(k * _L, _L)]
            return 0

        lax.fori_loop(0, nseg // _L, add, 0)
        return 0

    lax.fori_loop(1, 8, red, 0)

    def nrm(k, _):
        accb[pl.ds(k * _L, _L)] = _rsqrt16(accb[pl.ds(k * _L, _L)])
        return 0

    lax.fori_loop(0, nseg // _L, nrm, 0)
    pltpu.sync_copy(accb, norms_hbm.at[pl.ds(a * _NPAD + base, nseg)])


def _agg_body(x_hbm, norms_hbm, src_hbm, dst_hbm, xs_hbm, out_hbm,
              srcb, dstb, rows0, rows1, nsb, acc, gs0, gs1):
    c = lax.axis_index("c")
    s = lax.axis_index("s")
    base_rows = c * (_NS * _CH) + s * _CH
    nrows = _NPAD // _NS

    pltpu.sync_copy(norms_hbm.at[pl.ds(2 * c * _NPAD + s * nrows, nrows)], nsb)

    def sc_chunk(k, _):
        pltpu.sync_copy(x_hbm.at[pl.ds(s * nrows + k * 128, 128)], rows0)

        def rowgrp(g, __):
            wv = nsb[pl.ds(k * 128 + g * _L, _L)]
            for j in range(_L):
                r = g * _L + j
                w = wv[j]
                for v in range(_D // _L):
                    rows0[r, pl.ds(v * _L, _L)] = rows0[r, pl.ds(v * _L, _L)] * w
            return 0

        lax.fori_loop(0, 128 // _L, rowgrp, 0)
        pltpu.sync_copy(rows0, xs_hbm.at[pl.ds(c * _NPAD + s * nrows + k * 128, 128)])
        return 0

    lax.fori_loop(0, nrows // 128, sc_chunk, 0)

    zeros16 = jnp.zeros((_L,), jnp.float32)

    def z(t, _):
        rows0[t // 8, pl.ds((t % 8) * _L, _L)] = zeros16
        return 0

    lax.fori_loop(0, 1024, z, 0)

    def zc(k, _):
        pltpu.sync_copy(rows0, acc.at[pl.ds(s * nrows + k * 128, 128)])
        return 0

    lax.fori_loop(0, nrows // 128, zc, 0)
    plsc.subcore_barrier()

    def g_start(j, buf, sem):
        pltpu.async_copy(xs_hbm.at[srcb.at[j]], buf, sem)

    def g_wait(j, buf, sem):
        pltpu.make_async_copy(xs_hbm.at[srcb.at[j]], buf, sem).wait()

    def s_add(j, buf):
        pltpu.sync_copy(buf, acc.at[dstb.at[j]], add=True)

    def group(g, _):
        pltpu.sync_copy(src_hbm.at[pl.ds(base_rows + g * _GC, _GC)], srcb)
        pltpu.sync_copy(dst_hbm.at[pl.ds(base_rows + g * _GC, _GC)], dstb)
        g_start(0, rows0, gs0)

        def body(i, __):
            j0 = 2 * i
            g_start(j0 + 1, rows1, gs1)
            g_wait(j0, rows0, gs0)
            s_add(j0, rows0)

            @pl.when(i < _GC // 2 - 1)
            def _():
                g_start(j0 + 2, rows0, gs0)

            g_wait(j0 + 1, rows1, gs1)
            s_add(j0 + 1, rows1)
            return 0

        lax.fori_loop(0, _GC // 2, body, 0)
        return 0

    lax.fori_loop(0, _CH // _GC, group, 0)
    plsc.subcore_barrier()

    def dr(k, _):
        pltpu.sync_copy(acc.at[pl.ds(s * nrows + k * 128, 128)],
                        out_hbm.at[pl.ds(c * _NPAD + s * nrows + k * 128, 128)])
        return 0

    lax.fori_loop(0, nrows // 128, dr, 0)


_sc_params = pltpu.CompilerParams(needs_layout_passes=False)

_deg_kernel = pl.kernel(
    _deg_body,
    out_type=jax.ShapeDtypeStruct((4 * _NPAD,), jnp.float32),
    mesh=_mesh,
    compiler_params=_sc_params,
    scratch_types=[
        pltpu.VMEM((_NPAD,), jnp.float32),
        pltpu.VMEM((_SL,), jnp.int32),
        pltpu.VMEM((_NPAD // 8,), jnp.float32),
        pltpu.VMEM((_NPAD // 8,), jnp.float32),
        pltpu.VMEM_SHARED((_NS, _NPAD), jnp.float32),
    ],
)

_agg_kernel = pl.kernel(
    _agg_body,
    out_type=(
        jax.ShapeDtypeStruct((2 * _NPAD, _D), jnp.float32),
        jax.ShapeDtypeStruct((2 * _NPAD, _D), jnp.float32),
    ),
    mesh=_mesh,
    compiler_params=_sc_params,
    scratch_types=[
        pltpu.VMEM((_GC, 128), jnp.int32),
        pltpu.VMEM((_GC, 128), jnp.int32),
        pltpu.VMEM((128, _D), jnp.float32),
        pltpu.VMEM((128, _D), jnp.float32),
        pltpu.VMEM((_NPAD // _NS,), jnp.float32),
        pltpu.VMEM_SHARED((_NPAD, _D), jnp.float32),
        pltpu.SemaphoreType.DMA,
        pltpu.SemaphoreType.DMA,
    ],
)


_BR = 1024
_SUB = _BR // 128


def _colmat(row):
    ones = jnp.ones((1, 128), jnp.float32)
    return lax.dot_general(row, ones, (((0,), (0,)), ((), ())),
                           preferred_element_type=jnp.float32)


def _final_body(af_ref, al_ref, nf_ref, nl_ref, wf_ref, wl_ref, bf_ref, bl_ref, o_ref):
    mf = jnp.concatenate([_colmat(nf_ref[0, r:r + 1, :]) for r in range(_SUB)], axis=0)
    ml = jnp.concatenate([_colmat(nl_ref[0, r:r + 1, :]) for r in range(_SUB)], axis=0)
    h = (jnp.dot(af_ref[...] * mf, wf_ref[...], preferred_element_type=jnp.float32)
         + jnp.dot(al_ref[...] * ml, wl_ref[...], preferred_element_type=jnp.float32)
         + bf_ref[...] + bl_ref[...])
    o_ref[...] = jnp.maximum(h, 0.0)


_GB = _NPAD // _BR


def _final_call(agg, norms3, wf, wl, bf2, bl2):
    return pl.pallas_call(
        _final_body,
        grid=(_GB,),
        in_specs=[
            pl.BlockSpec((_BR, _D), lambda i: (i, 0)),
            pl.BlockSpec((_BR, _D), lambda i: (i + _GB, 0)),
            pl.BlockSpec((1, _SUB, 128), lambda i: (_GB + i, 0, 0)),
            pl.BlockSpec((1, _SUB, 128), lambda i: (3 * _GB + i, 0, 0)),
            pl.BlockSpec((_D, _D), lambda i: (0, 0)),
            pl.BlockSpec((_D, _D), lambda i: (0, 0)),
            pl.BlockSpec((1, 128), lambda i: (0, 0)),
            pl.BlockSpec((1, 128), lambda i: (0, 0)),
        ],
        out_specs=pl.BlockSpec((_BR, _D), lambda i: (i, 0)),
        out_shape=jax.ShapeDtypeStruct((_N, _D), jnp.float32),
    )(agg, agg, norms3, norms3, wf, wl, bf2, bl2)


def kernel(x, edge_index_follows, edge_index_likes, W_follows, b_follows, W_likes, b_likes):
    i32 = jnp.int32
    x_pad = jnp.concatenate([x, jnp.zeros((_NPAD - _N, _D), x.dtype)], axis=0)
    pad = jnp.full((_EP - _E,), _N, i32)
    srcf = jnp.concatenate([edge_index_follows[0], pad])
    dstf = jnp.concatenate([edge_index_follows[1], pad])
    srcl = jnp.concatenate([edge_index_likes[0], pad])
    dstl = jnp.concatenate([edge_index_likes[1], pad])

    idx_all = jnp.concatenate([srcf, dstf, srcl, dstl])
    norms = _deg_kernel(idx_all)

    src_c = jnp.concatenate([srcf, srcl + _NPAD]).reshape(2 * _NS * _CH, 128)
    dst_c = jnp.concatenate([dstf, dstl]).reshape(2 * _NS * _CH, 128)
    _, agg = _agg_kernel(x_pad, norms, src_c, dst_c)

    out = _final_call(agg, norms.reshape(4 * _GB, _SUB, 128), W_follows, W_likes,
                      b_follows.reshape(1, _D), b_likes.reshape(1, _D))
    return out


# speedup vs baseline: 1.3988x; 1.0015x over previous
import functools

import jax
import jax.numpy as jnp
from jax import lax
from jax.experimental import pallas as pl
from jax.experimental.pallas import tpu as pltpu
from jax.experimental.pallas import tpu_sc as plsc

_N = 10000
_D = 128
_E = 160000
_NPAD = 10240
_NC = 2
_NS = 16
_L = 16
_CH = 80
_TE = _CH * 128
_EP = _TE * _NS
_SL = _EP // 8
_GC = 40

_mesh = plsc.VectorSubcoreMesh(
    core_axis_name="c", subcore_axis_name="s", num_cores=_NC, num_subcores=_NS
)


def _rsqrt16(d):
    i = plsc.bitcast(d, jnp.int32)
    magic = jnp.full((_L,), 0x5F3759DF, jnp.int32)
    one = jnp.full((_L,), 1, jnp.int32)
    y = plsc.bitcast(magic - lax.shift_right_logical(i, one), jnp.float32)
    h = 0.5 * d
    y = y * (1.5 - h * y * y)
    y = y * (1.5 - h * y * y)
    y = y * (1.5 - h * y * y)
    return jnp.where(d > 0.0, y, 0.0)


def _deg_body(idx_hbm, norms_hbm, deg, idxb, accb, tmpb, part):
    c = lax.axis_index("c")
    s = lax.axis_index("s")
    a = 2 * c + s // 8
    slot = s % 8
    zeros16 = jnp.zeros((_L,), jnp.float32)
    ones16 = jnp.ones((_L,), jnp.float32)

    def z(k, _):
        deg[pl.ds(k * _L, _L)] = zeros16
        return 0

    lax.fori_loop(0, _NPAD // _L, z, 0)

    pltpu.sync_copy(idx_hbm.at[pl.ds(a * _EP + slot * _SL, _SL)], idxb)

    def cnt(k, _):
        iv = idxb[pl.ds(k * _L, _L)]
        plsc.addupdate_scatter(deg, [iv], ones16)
        return 0

    lax.fori_loop(0, _SL // _L, cnt, 0)

    pltpu.sync_copy(deg, part.at[s])
    plsc.subcore_barrier()

    nseg = _NPAD // 8
    base = slot * nseg
    row0 = (s // 8) * 8
    pltpu.sync_copy(part.at[row0, pl.ds(base, nseg)], accb)

    def red(r, _):
        pltpu.sync_copy(part.at[row0 + r, pl.ds(base, nseg)], tmpb)

        def add(k, __):
            accb[pl.ds(k * _L, _L)] = accb[pl.ds(k * _L, _L)] + tmpb[pl.ds(k * _L, _L)]
            return 0

        lax.fori_loop(0, nseg // _L, add, 0)
        return 0

    lax.fori_loop(1, 8, red, 0)

    def nrm(k, _):
        accb[pl.ds(k * _L, _L)] = _rsqrt16(accb[pl.ds(k * _L, _L)])
        return 0

    lax.fori_loop(0, nseg // _L, nrm, 0)
    pltpu.sync_copy(accb, norms_hbm.at[pl.ds(a * _NPAD + base, nseg)])


def _agg_body(x_hbm, norms_hbm, src_hbm, dst_hbm, xs_hbm, out_hbm,
              srcb, dstb, rows0, rows1, nsb, acc, gs0, gs1):
    c = lax.axis_index("c")
    s = lax.axis_index("s")
    base_rows = c * (_NS * _CH) + s * _CH
    nrows = _NPAD // _NS

    pltpu.sync_copy(norms_hbm.at[pl.ds(2 * c * _NPAD + s * nrows, nrows)], nsb)

    def sc_chunk(k, _):
        pltpu.sync_copy(x_hbm.at[pl.ds(s * nrows + k * 128, 128)], rows0)

        def rowgrp(g, __):
            wv = nsb[pl.ds(k * 128 + g * _L, _L)]
            for j in range(_L):
                r = g * _L + j
                w = wv[j]
                for v in range(_D // _L):
                    rows0[r, pl.ds(v * _L, _L)] = rows0[r, pl.ds(v * _L, _L)] * w
            return 0

        lax.fori_loop(0, 128 // _L, rowgrp, 0)
        pltpu.sync_copy(rows0, xs_hbm.at[pl.ds(c * _NPAD + s * nrows + k * 128, 128)])
        return 0

    lax.fori_loop(0, nrows // 128, sc_chunk, 0)

    zeros16 = jnp.zeros((_L,), jnp.float32)

    def z(t, _):
        rows0[t // 8, pl.ds((t % 8) * _L, _L)] = zeros16
        return 0

    lax.fori_loop(0, 1024, z, 0)

    def zc(k, _):
        pltpu.sync_copy(rows0, acc.at[pl.ds(s * nrows + k * 128, 128)])
        return 0

    lax.fori_loop(0, nrows // 128, zc, 0)
    plsc.subcore_barrier()

    def g_start(j, buf, sem):
        pltpu.async_copy(xs_hbm.at[srcb.at[j, pl.ds(0, 64)]],
                         buf.at[pl.ds(0, 64)], sem)
        pltpu.async_copy(xs_hbm.at[srcb.at[j, pl.ds(64, 64)]],
                         buf.at[pl.ds(64, 64)], sem)

    def g_wait(j, buf, sem):
        pltpu.make_async_copy(xs_hbm.at[srcb.at[j]], buf, sem).wait()

    def s_add(j, buf):
        pltpu.sync_copy(buf, acc.at[dstb.at[j]], add=True)

    def group(g, _):
        pltpu.sync_copy(src_hbm.at[pl.ds(base_rows + g * _GC, _GC)], srcb)
        pltpu.sync_copy(dst_hbm.at[pl.ds(base_rows + g * _GC, _GC)], dstb)
        g_start(0, rows0, gs0)

        def body(i, __):
            j0 = 2 * i
            g_start(j0 + 1, rows1, gs1)
            g_wait(j0, rows0, gs0)
            s_add(j0, rows0)

            @pl.when(i < _GC // 2 - 1)
            def _():
                g_start(j0 + 2, rows0, gs0)

            g_wait(j0 + 1, rows1, gs1)
            s_add(j0 + 1, rows1)
            return 0

        lax.fori_loop(0, _GC // 2, body, 0)
        return 0

    lax.fori_loop(0, _CH // _GC, group, 0)
    plsc.subcore_barrier()

    def dr(k, _):
        pltpu.sync_copy(acc.at[pl.ds(s * nrows + k * 128, 128)],
                        out_hbm.at[pl.ds(c * _NPAD + s * nrows + k * 128, 128)])
        return 0

    lax.fori_loop(0, nrows // 128, dr, 0)


_sc_params = pltpu.CompilerParams(needs_layout_passes=False)

_deg_kernel = pl.kernel(
    _deg_body,
    out_type=jax.ShapeDtypeStruct((4 * _NPAD,), jnp.float32),
    mesh=_mesh,
    compiler_params=_sc_params,
    scratch_types=[
        pltpu.VMEM((_NPAD,), jnp.float32),
        pltpu.VMEM((_SL,), jnp.int32),
        pltpu.VMEM((_NPAD // 8,), jnp.float32),
        pltpu.VMEM((_NPAD // 8,), jnp.float32),
        pltpu.VMEM_SHARED((_NS, _NPAD), jnp.float32),
    ],
)

_agg_kernel = pl.kernel(
    _agg_body,
    out_type=(
        jax.ShapeDtypeStruct((2 * _NPAD, _D), jnp.float32),
        jax.ShapeDtypeStruct((2 * _NPAD, _D), jnp.float32),
    ),
    mesh=_mesh,
    compiler_params=_sc_params,
    scratch_types=[
        pltpu.VMEM((_GC, 128), jnp.int32),
        pltpu.VMEM((_GC, 128), jnp.int32),
        pltpu.VMEM((128, _D), jnp.float32),
        pltpu.VMEM((128, _D), jnp.float32),
        pltpu.VMEM((_NPAD // _NS,), jnp.float32),
        pltpu.VMEM_SHARED((_NPAD, _D), jnp.float32),
        pltpu.SemaphoreType.DMA,
        pltpu.SemaphoreType.DMA,
    ],
)


_BR = 1024
_SUB = _BR // 128


def _colmat(row):
    ones = jnp.ones((1, 128), jnp.float32)
    return lax.dot_general(row, ones, (((0,), (0,)), ((), ())),
                           preferred_element_type=jnp.float32)


def _final_body(af_ref, al_ref, nf_ref, nl_ref, wf_ref, wl_ref, bf_ref, bl_ref, o_ref):
    mf = jnp.concatenate([_colmat(nf_ref[0, r:r + 1, :]) for r in range(_SUB)], axis=0)
    ml = jnp.concatenate([_colmat(nl_ref[0, r:r + 1, :]) for r in range(_SUB)], axis=0)
    h = (jnp.dot(af_ref[...] * mf, wf_ref[...], preferred_element_type=jnp.float32)
         + jnp.dot(al_ref[...] * ml, wl_ref[...], preferred_element_type=jnp.float32)
         + bf_ref[...] + bl_ref[...])
    o_ref[...] = jnp.maximum(h, 0.0)


_GB = _NPAD // _BR


def _final_call(agg, norms3, wf, wl, bf2, bl2):
    return pl.pallas_call(
        _final_body,
        grid=(_GB,),
        in_specs=[
            pl.BlockSpec((_BR, _D), lambda i: (i, 0)),
            pl.BlockSpec((_BR, _D), lambda i: (i + _GB, 0)),
            pl.BlockSpec((1, _SUB, 128), lambda i: (_GB + i, 0, 0)),
            pl.BlockSpec((1, _SUB, 128), lambda i: (3 * _GB + i, 0, 0)),
            pl.BlockSpec((_D, _D), lambda i: (0, 0)),
            pl.BlockSpec((_D, _D), lambda i: (0, 0)),
            pl.BlockSpec((1, 128), lambda i: (0, 0)),
            pl.BlockSpec((1, 128), lambda i: (0, 0)),
        ],
        out_specs=pl.BlockSpec((_BR, _D), lambda i: (i, 0)),
        out_shape=jax.ShapeDtypeStruct((_N, _D), jnp.float32),
    )(agg, agg, norms3, norms3, wf, wl, bf2, bl2)


def kernel(x, edge_index_follows, edge_index_likes, W_follows, b_follows, W_likes, b_likes):
    i32 = jnp.int32
    x_pad = jnp.concatenate([x, jnp.zeros((_NPAD - _N, _D), x.dtype)], axis=0)
    pad = jnp.full((_EP - _E,), _N, i32)
    srcf = jnp.concatenate([edge_index_follows[0], pad])
    dstf = jnp.concatenate([edge_index_follows[1], pad])
    srcl = jnp.concatenate([edge_index_likes[0], pad])
    dstl = jnp.concatenate([edge_index_likes[1], pad])

    idx_all = jnp.concatenate([srcf, dstf, srcl, dstl])
    norms = _deg_kernel(idx_all)

    src_c = jnp.concatenate([srcf, srcl + _NPAD]).reshape(2 * _NS * _CH, 128)
    dst_c = jnp.concatenate([dstf, dstl]).reshape(2 * _NS * _CH, 128)
    _, agg = _agg_kernel(x_pad, norms, src_c, dst_c)

    out = _final_call(agg, norms.reshape(4 * _GB, _SUB, 128), W_follows, W_likes,
                      b_follows.reshape(1, _D), b_likes.reshape(1, _D))
    return out


# speedup vs baseline: 1.4137x; 1.0106x over previous
import functools

import jax
import jax.numpy as jnp
from jax import lax
from jax.experimental import pallas as pl
from jax.experimental.pallas import tpu as pltpu
from jax.experimental.pallas import tpu_sc as plsc

_N = 10000
_D = 128
_E = 160000
_NPAD = 10240
_NC = 2
_NS = 16
_L = 16
_CH = 80
_TE = _CH * 128
_EP = _TE * _NS
_SL = _EP // 8
_GC = 40

_mesh = plsc.VectorSubcoreMesh(
    core_axis_name="c", subcore_axis_name="s", num_cores=_NC, num_subcores=_NS
)


def _rsqrt16(d):
    i = plsc.bitcast(d, jnp.int32)
    magic = jnp.full((_L,), 0x5F3759DF, jnp.int32)
    one = jnp.full((_L,), 1, jnp.int32)
    y = plsc.bitcast(magic - lax.shift_right_logical(i, one), jnp.float32)
    h = 0.5 * d
    y = y * (1.5 - h * y * y)
    y = y * (1.5 - h * y * y)
    y = y * (1.5 - h * y * y)
    return jnp.where(d > 0.0, y, 0.0)


def _deg_body(idx_hbm, norms_hbm, deg, idxb, accb, tmpb, part):
    c = lax.axis_index("c")
    s = lax.axis_index("s")
    a = 2 * c + s // 8
    slot = s % 8
    zeros16 = jnp.zeros((_L,), jnp.float32)
    ones16 = jnp.ones((_L,), jnp.float32)

    def z(k, _):
        for u in range(4):
            deg[pl.ds((4 * k + u) * _L, _L)] = zeros16
        return 0

    lax.fori_loop(0, _NPAD // _L // 4, z, 0)

    pltpu.sync_copy(idx_hbm.at[pl.ds(a * _EP + slot * _SL, _SL)], idxb)

    def cnt(k, _):
        for u in range(8):
            iv = idxb[pl.ds((8 * k + u) * _L, _L)]
            plsc.addupdate_scatter(deg, [iv], ones16)
        return 0

    lax.fori_loop(0, _SL // _L // 8, cnt, 0)

    pltpu.sync_copy(deg, part.at[s])
    plsc.subcore_barrier()

    nseg = _NPAD // 8
    base = slot * nseg
    row0 = (s // 8) * 8
    pltpu.sync_copy(part.at[row0, pl.ds(base, nseg)], accb)

    def red(r, _):
        pltpu.sync_copy(part.at[row0 + r, pl.ds(base, nseg)], tmpb)

        def add(k, __):
            for u in range(4):
                q = pl.ds((4 * k + u) * _L, _L)
                accb[q] = accb[q] + tmpb[q]
            return 0

        lax.fori_loop(0, nseg // _L // 4, add, 0)
        return 0

    lax.fori_loop(1, 8, red, 0)

    def nrm(k, _):
        accb[pl.ds(k * _L, _L)] = _rsqrt16(accb[pl.ds(k * _L, _L)])
        return 0

    lax.fori_loop(0, nseg // _L, nrm, 0)
    pltpu.sync_copy(accb, norms_hbm.at[pl.ds(a * _NPAD + base, nseg)])


def _agg_body(x_hbm, norms_hbm, src_hbm, dst_hbm, xs_hbm, out_hbm,
              srcb, dstb, rows0, rows1, nsb, acc, gs0, gs1):
    c = lax.axis_index("c")
    s = lax.axis_index("s")
    base_rows = c * (_NS * _CH) + s * _CH
    nrows = _NPAD // _NS

    pltpu.sync_copy(norms_hbm.at[pl.ds(2 * c * _NPAD + s * nrows, nrows)], nsb)

    def sc_chunk(k, _):
        pltpu.sync_copy(x_hbm.at[pl.ds(s * nrows + k * 128, 128)], rows0)

        def rowgrp(g, __):
            wv = nsb[pl.ds(k * 128 + g * _L, _L)]
            for j in range(_L):
                r = g * _L + j
                w = wv[j]
                for v in range(_D // _L):
                    rows0[r, pl.ds(v * _L, _L)] = rows0[r, pl.ds(v * _L, _L)] * w
            return 0

        lax.fori_loop(0, 128 // _L, rowgrp, 0)
        pltpu.sync_copy(rows0, xs_hbm.at[pl.ds(c * _NPAD + s * nrows + k * 128, 128)])
        return 0

    lax.fori_loop(0, nrows // 128, sc_chunk, 0)

    zeros16 = jnp.zeros((_L,), jnp.float32)

    def z(t, _):
        rows0[t // 8, pl.ds((t % 8) * _L, _L)] = zeros16
        return 0

    lax.fori_loop(0, 1024, z, 0)

    def zc(k, _):
        pltpu.sync_copy(rows0, acc.at[pl.ds(s * nrows + k * 128, 128)])
        return 0

    lax.fori_loop(0, nrows // 128, zc, 0)
    plsc.subcore_barrier()

    def g_start(j, buf, sem):
        pltpu.async_copy(xs_hbm.at[srcb.at[j]], buf, sem)

    def g_wait(j, buf, sem):
        pltpu.make_async_copy(xs_hbm.at[srcb.at[j]], buf, sem).wait()

    def s_add(j, buf):
        pltpu.sync_copy(buf, acc.at[dstb.at[j]], add=True)

    def group(g, _):
        pltpu.sync_copy(src_hbm.at[pl.ds(base_rows + g * _GC, _GC)], srcb)
        pltpu.sync_copy(dst_hbm.at[pl.ds(base_rows + g * _GC, _GC)], dstb)
        g_start(0, rows0, gs0)

        def body(i, __):
            j0 = 2 * i
            g_start(j0 + 1, rows1, gs1)
            g_wait(j0, rows0, gs0)
            s_add(j0, rows0)

            @pl.when(i < _GC // 2 - 1)
            def _():
                g_start(j0 + 2, rows0, gs0)

            g_wait(j0 + 1, rows1, gs1)
            s_add(j0 + 1, rows1)
            return 0

        lax.fori_loop(0, _GC // 2, body, 0)
        return 0

    lax.fori_loop(0, _CH // _GC, group, 0)
    plsc.subcore_barrier()

    def dr(k, _):
        pltpu.sync_copy(acc.at[pl.ds(s * nrows + k * 128, 128)],
                        out_hbm.at[pl.ds(c * _NPAD + s * nrows + k * 128, 128)])
        return 0

    lax.fori_loop(0, nrows // 128, dr, 0)


_sc_params = pltpu.CompilerParams(needs_layout_passes=False)

_deg_kernel = pl.kernel(
    _deg_body,
    out_type=jax.ShapeDtypeStruct((4 * _NPAD,), jnp.float32),
    mesh=_mesh,
    compiler_params=_sc_params,
    scratch_types=[
        pltpu.VMEM((_NPAD,), jnp.float32),
        pltpu.VMEM((_SL,), jnp.int32),
        pltpu.VMEM((_NPAD // 8,), jnp.float32),
        pltpu.VMEM((_NPAD // 8,), jnp.float32),
        pltpu.VMEM_SHARED((_NS, _NPAD), jnp.float32),
    ],
)

_agg_kernel = pl.kernel(
    _agg_body,
    out_type=(
        jax.ShapeDtypeStruct((2 * _NPAD, _D), jnp.float32),
        jax.ShapeDtypeStruct((2 * _NPAD, _D), jnp.float32),
    ),
    mesh=_mesh,
    compiler_params=_sc_params,
    scratch_types=[
        pltpu.VMEM((_GC, 128), jnp.int32),
        pltpu.VMEM((_GC, 128), jnp.int32),
        pltpu.VMEM((128, _D), jnp.float32),
        pltpu.VMEM((128, _D), jnp.float32),
        pltpu.VMEM((_NPAD // _NS,), jnp.float32),
        pltpu.VMEM_SHARED((_NPAD, _D), jnp.float32),
        pltpu.SemaphoreType.DMA,
        pltpu.SemaphoreType.DMA,
    ],
)


_BR = 1024
_SUB = _BR // 128


def _colmat(row):
    ones = jnp.ones((1, 128), jnp.float32)
    return lax.dot_general(row, ones, (((0,), (0,)), ((), ())),
                           preferred_element_type=jnp.float32)


def _final_body(af_ref, al_ref, nf_ref, nl_ref, wf_ref, wl_ref, bf_ref, bl_ref, o_ref):
    mf = jnp.concatenate([_colmat(nf_ref[0, r:r + 1, :]) for r in range(_SUB)], axis=0)
    ml = jnp.concatenate([_colmat(nl_ref[0, r:r + 1, :]) for r in range(_SUB)], axis=0)
    h = (jnp.dot(af_ref[...] * mf, wf_ref[...], preferred_element_type=jnp.float32)
         + jnp.dot(al_ref[...] * ml, wl_ref[...], preferred_element_type=jnp.float32)
         + bf_ref[...] + bl_ref[...])
    o_ref[...] = jnp.maximum(h, 0.0)


_GB = _NPAD // _BR


def _final_call(agg, norms3, wf, wl, bf2, bl2):
    return pl.pallas_call(
        _final_body,
        grid=(_GB,),
        in_specs=[
            pl.BlockSpec((_BR, _D), lambda i: (i, 0)),
            pl.BlockSpec((_BR, _D), lambda i: (i + _GB, 0)),
            pl.BlockSpec((1, _SUB, 128), lambda i: (_GB + i, 0, 0)),
            pl.BlockSpec((1, _SUB, 128), lambda i: (3 * _GB + i, 0, 0)),
            pl.BlockSpec((_D, _D), lambda i: (0, 0)),
            pl.BlockSpec((_D, _D), lambda i: (0, 0)),
            pl.BlockSpec((1, 128), lambda i: (0, 0)),
            pl.BlockSpec((1, 128), lambda i: (0, 0)),
        ],
        out_specs=pl.BlockSpec((_BR, _D), lambda i: (i, 0)),
        out_shape=jax.ShapeDtypeStruct((_N, _D), jnp.float32),
    )(agg, agg, norms3, norms3, wf, wl, bf2, bl2)


def kernel(x, edge_index_follows, edge_index_likes, W_follows, b_follows, W_likes, b_likes):
    i32 = jnp.int32
    x_pad = jnp.concatenate([x, jnp.zeros((_NPAD - _N, _D), x.dtype)], axis=0)
    pad = jnp.full((_EP - _E,), _N, i32)
    srcf = jnp.concatenate([edge_index_follows[0], pad])
    dstf = jnp.concatenate([edge_index_follows[1], pad])
    srcl = jnp.concatenate([edge_index_likes[0], pad])
    dstl = jnp.concatenate([edge_index_likes[1], pad])

    idx_all = jnp.concatenate([srcf, dstf, srcl, dstl])
    norms = _deg_kernel(idx_all)

    src_c = jnp.concatenate([srcf, srcl + _NPAD]).reshape(2 * _NS * _CH, 128)
    dst_c = jnp.concatenate([dstf, dstl]).reshape(2 * _NS * _CH, 128)
    _, agg = _agg_kernel(x_pad, norms, src_c, dst_c)

    out = _final_call(agg, norms.reshape(4 * _GB, _SUB, 128), W_follows, W_likes,
                      b_follows.reshape(1, _D), b_likes.reshape(1, _D))
    return out
